# Initial kernel scaffold; baseline (speedup 1.0000x reference)
import functools

import jax
import jax.numpy as jnp
from jax import lax
from jax.experimental import pallas as pl
from jax.experimental.pallas import tpu as pltpu
from jax.experimental.pallas import tpu_sc as plsc

CUTOFF = 12.0
CUTOFF2 = CUTOFF ** 2
KE = 7.199822675975274
CUTON = 0.8 * CUTOFF

_S0 = 3.14159264
_S1 = -5.1677107
_S2 = 2.5500919
_S3 = -0.59839523
_S4 = 0.07788843

NC = 2
NS = 16
L = 16
NW = NC * NS


def _switch(d):
    x = (d - CUTON) * (1.0 / (CUTOFF - CUTON))
    x = jnp.minimum(jnp.maximum(x, 0.0), 1.0)
    t = x - 0.5
    u = t * t
    p = _S4
    p = p * u + _S3
    p = p * u + _S2
    p = p * u + _S1
    p = p * u + _S0
    s = p * t
    return 0.5 * (1.0 - s)


def _make_kernel(E, N, CH):
    n_chunks_total = E // CH
    per_w = n_chunks_total // NW
    mesh = plsc.VectorSubcoreMesh(core_axis_name="c", subcore_axis_name="s")

    @functools.partial(
        pl.kernel,
        out_type=jax.ShapeDtypeStruct((E,), jnp.float32),
        mesh=mesh,
        scratch_types=[
            pltpu.VMEM((N,), jnp.float32),
            pltpu.VMEM((CH,), jnp.int32),
            pltpu.VMEM((CH,), jnp.int32),
            pltpu.VMEM((CH,), jnp.float32),
            pltpu.VMEM((CH,), jnp.float32),
        ],
    )
    def k(dist_hbm, q_hbm, idxu_hbm, idxv_hbm, out_hbm, q_v, u_v, v_v, d_v, o_v):
        wid = lax.axis_index("s") * NC + lax.axis_index("c")
        base = wid * (per_w * CH)
        pltpu.sync_copy(q_hbm, q_v)

        def chunk(ci, _):
            off = base + ci * CH
            pltpu.sync_copy(idxu_hbm.at[pl.ds(off, CH)], u_v)
            pltpu.sync_copy(idxv_hbm.at[pl.ds(off, CH)], v_v)
            pltpu.sync_copy(dist_hbm.at[pl.ds(off, CH)], d_v)

            def vec(i, _):
                s = i * L
                qi = plsc.load_gather(q_v, [u_v[pl.ds(s, L)]])
                qj = plsc.load_gather(q_v, [v_v[pl.ds(s, L)]])
                d = d_v[pl.ds(s, L)]
                chi = 1.0 / d
                chi_shift = (2.0 / CUTOFF) - d * (1.0 / CUTOFF2)
                o_v[pl.ds(s, L)] = (KE * qi) * qj * (chi - chi_shift) * _switch(d)
                return 0

            lax.fori_loop(0, CH // L, vec, 0)
            pltpu.sync_copy(o_v, out_hbm.at[pl.ds(off, CH)])
            return 0

        lax.fori_loop(0, per_w, chunk, 0)

    return k


def kernel(mlmm_distances, mlmm_atomic_charges, mlmm_idxu, mlmm_idxv):
    E = mlmm_distances.shape[0]
    N = mlmm_atomic_charges.shape[0]
    k = _make_kernel(E, N, 4000)
    return k(mlmm_distances, mlmm_atomic_charges,
             mlmm_idxu.astype(jnp.int32), mlmm_idxv.astype(jnp.int32))


# speedup vs baseline: 460.8342x; 460.8342x over previous
import functools

import jax
import jax.numpy as jnp
from jax import lax
from jax.experimental import pallas as pl
from jax.experimental.pallas import tpu as pltpu
from jax.experimental.pallas import tpu_sc as plsc

CUTOFF = 12.0
CUTOFF2 = CUTOFF ** 2
KE = 7.199822675975274
CUTON = 0.8 * CUTOFF

_S0 = 3.14159264
_S1 = -5.1677107
_S2 = 2.5500919
_S3 = -0.59839523
_S4 = 0.07788843

NC = 2
NS = 16
L = 16
NW = NC * NS


def _switch(d):
    x = (d - CUTON) * (1.0 / (CUTOFF - CUTON))
    x = jnp.minimum(jnp.maximum(x, 0.0), 1.0)
    t = x - 0.5
    u = t * t
    p = _S4
    p = p * u + _S3
    p = p * u + _S2
    p = p * u + _S1
    p = p * u + _S0
    s = p * t
    return 0.5 * (1.0 - s)


def _make_kernel(E, N, CH):
    n_chunks_total = E // CH
    per_w = n_chunks_total // NW
    mesh = plsc.VectorSubcoreMesh(core_axis_name="c", subcore_axis_name="s")

    @functools.partial(
        pl.kernel,
        out_type=jax.ShapeDtypeStruct((E,), jnp.float32),
        mesh=mesh,
        compiler_params=pltpu.CompilerParams(needs_layout_passes=False),
        scratch_types=[
            pltpu.VMEM((N,), jnp.float32),
            pltpu.VMEM((CH,), jnp.int32),
            pltpu.VMEM((CH,), jnp.int32),
            pltpu.VMEM((CH,), jnp.float32),
            pltpu.VMEM((CH,), jnp.float32),
        ],
    )
    def k(dist_hbm, q_hbm, idxu_hbm, idxv_hbm, out_hbm, q_v, u_v, v_v, d_v, o_v):
        wid = lax.axis_index("s") * NC + lax.axis_index("c")
        base = wid * (per_w * CH)
        pltpu.sync_copy(q_hbm, q_v)

        def chunk(ci, _):
            off = base + ci * CH
            pltpu.sync_copy(idxu_hbm.at[pl.ds(off, CH)], u_v)
            pltpu.sync_copy(idxv_hbm.at[pl.ds(off, CH)], v_v)
            pltpu.sync_copy(dist_hbm.at[pl.ds(off, CH)], d_v)

            def vec(i, _):
                s = i * L
                qi = plsc.load_gather(q_v, [u_v[pl.ds(s, L)]])
                qj = plsc.load_gather(q_v, [v_v[pl.ds(s, L)]])
                d = d_v[pl.ds(s, L)]
                chi = 1.0 / d
                chi_shift = (2.0 / CUTOFF) - d * (1.0 / CUTOFF2)
                o_v[pl.ds(s, L)] = (KE * qi) * qj * (chi - chi_shift) * _switch(d)
                return 0

            lax.fori_loop(0, CH // L, vec, 0)
            pltpu.sync_copy(o_v, out_hbm.at[pl.ds(off, CH)])
            return 0

        lax.fori_loop(0, per_w, chunk, 0)

    return k


def kernel(mlmm_distances, mlmm_atomic_charges, mlmm_idxu, mlmm_idxv):
    E = mlmm_distances.shape[0]
    N = mlmm_atomic_charges.shape[0]
    k = _make_kernel(E, N, 4000)
    return k(mlmm_distances, mlmm_atomic_charges,
             mlmm_idxu.astype(jnp.int32), mlmm_idxv.astype(jnp.int32))


# speedup vs baseline: 967.3852x; 2.0992x over previous
import functools

import jax
import jax.numpy as jnp
from jax import lax
from jax.experimental import pallas as pl
from jax.experimental.pallas import tpu as pltpu
from jax.experimental.pallas import tpu_sc as plsc

CUTOFF = 12.0
CUTON = 0.8 * CUTOFF
KE = 7.199822675975274

_IW = 1.0 / (CUTOFF - CUTON)
_T0 = -CUTON / (CUTOFF - CUTON) - 0.5
_KA = 3.599911337987637
_Q0 = -11.30876469
_Q1 = 18.54495726
_Q2 = -8.47143384
_NC2 = -1.0 / (CUTOFF * CUTOFF)
_C2A = 2.0 / CUTOFF

NC = 2
NS = 16
L = 16
NW = NC * NS


def _make_kernel(E, N, CH):
    per_w = E // (NW * CH)
    mesh = plsc.VectorSubcoreMesh(core_axis_name="c", subcore_axis_name="s")

    @functools.partial(
        pl.kernel,
        out_type=jax.ShapeDtypeStruct((E,), jnp.float32),
        mesh=mesh,
        compiler_params=pltpu.CompilerParams(needs_layout_passes=False),
        scratch_types=[
            pltpu.VMEM((N,), jnp.float32),
            pltpu.VMEM((CH,), jnp.int32),
            pltpu.VMEM((CH,), jnp.int32),
            pltpu.VMEM((CH,), jnp.int32),
            pltpu.VMEM((CH,), jnp.int32),
            pltpu.VMEM((CH,), jnp.float32),
            pltpu.VMEM((CH,), jnp.float32),
            pltpu.VMEM((CH,), jnp.float32),
            pltpu.VMEM((CH,), jnp.float32),
            pltpu.SemaphoreType.DMA,
            pltpu.SemaphoreType.DMA,
            pltpu.SemaphoreType.DMA,
            pltpu.SemaphoreType.DMA,
        ],
    )
    def k(dist_hbm, q_hbm, idxu_hbm, idxv_hbm, out_hbm,
          q_v, u0, u1, v0, v1, d0, d1, o0, o1, si0, si1, so0, so1):
        wid = lax.axis_index("s") * NC + lax.axis_index("c")
        base = wid * (per_w * CH)
        ubuf = (u0, u1)
        vbuf = (v0, v1)
        dbuf = (d0, d1)
        obuf = (o0, o1)
        sin_ = (si0, si1)
        sout = (so0, so1)

        pltpu.sync_copy(q_hbm, q_v)

        def issue_in(ci, b):
            off = base + ci * CH
            pltpu.async_copy(idxu_hbm.at[pl.ds(off, CH)], ubuf[b], sin_[b])
            pltpu.async_copy(idxv_hbm.at[pl.ds(off, CH)], vbuf[b], sin_[b])
            pltpu.async_copy(dist_hbm.at[pl.ds(off, CH)], dbuf[b], sin_[b])

        def wait_in(b):
            pltpu.make_async_copy(idxu_hbm.at[pl.ds(0, CH)], ubuf[b], sin_[b]).wait()
            pltpu.make_async_copy(idxv_hbm.at[pl.ds(0, CH)], vbuf[b], sin_[b]).wait()
            pltpu.make_async_copy(dist_hbm.at[pl.ds(0, CH)], dbuf[b], sin_[b]).wait()

        def compute(b):
            @plsc.parallel_loop(0, CH // L, unroll=8)
            def _(i):
                s = i * L
                d = dbuf[b][pl.ds(s, L)]
                qi = plsc.load_gather(q_v, [ubuf[b][pl.ds(s, L)]])
                qj = plsc.load_gather(q_v, [vbuf[b][pl.ds(s, L)]])
                t = jnp.minimum(jnp.maximum(d * _IW + _T0, -0.5), 0.5)
                u = t * t
                p = _Q2 * u + _Q1
                p = p * u + _Q0
                w = t * p + _KA
                g = 1.0 / d - (d * _NC2 + _C2A)
                obuf[b][pl.ds(s, L)] = (qi * qj) * g * w

        def outer(cc, _):
            for b in (0, 1):
                ci = cc * 2 + b
                wait_in(b)
                if b == 0:
                    issue_in(ci + 1, 1)
                else:
                    @pl.when(cc < per_w // 2 - 1)
                    def _():
                        issue_in(ci + 1, 0)

                @pl.when(cc >= 1)
                def _():
                    pltpu.make_async_copy(
                        obuf[b], out_hbm.at[pl.ds(0, CH)], sout[b]).wait()

                compute(b)
                off = base + ci * CH
                pltpu.async_copy(obuf[b], out_hbm.at[pl.ds(off, CH)], sout[b])
            return 0

        issue_in(0, 0)
        lax.fori_loop(0, per_w // 2, outer, 0)
        for b in (0, 1):
            pltpu.make_async_copy(obuf[b], out_hbm.at[pl.ds(0, CH)], sout[b]).wait()

    return k


def kernel(mlmm_distances, mlmm_atomic_charges, mlmm_idxu, mlmm_idxv):
    E = mlmm_distances.shape[0]
    N = mlmm_atomic_charges.shape[0]
    k = _make_kernel(E, N, 2000)
    return k(mlmm_distances, mlmm_atomic_charges,
             mlmm_idxu.astype(jnp.int32), mlmm_idxv.astype(jnp.int32))
